# baseline (speedup 1.0000x reference)
import functools

import jax
import jax.numpy as jnp
from jax import lax
from jax.experimental import pallas as pl
from jax.experimental.pallas import tpu as pltpu
from jax.experimental.pallas import tpu_sc as plsc

B = 16384
EMB = 64
IN_DIM = 3 * EMB
H1 = 128
H2 = 64

_HALF_U = 501760
_BC_U = 6272
_HALF_I = 51200
_BC_I = 10240
_HALF_G = 512
_BC_G = 512

_NC = 2
_NS = 16
_NW = _NC * _NS
_BPW = B // _NW
_CHUNK = 128
_NCHUNK = _BPW // _CHUNK


def _pack_body(l_ref, r_ref, o_ref):
    o_ref[...] = jnp.concatenate([l_ref[...].T, r_ref[...].T], axis=1)


def _pack(table_t, half, bc):
    nb = half // bc
    return pl.pallas_call(
        _pack_body,
        grid=(nb,),
        in_specs=[
            pl.BlockSpec((EMB, bc), lambda n: (0, n)),
            pl.BlockSpec((EMB, bc), lambda n, _nb=nb: (0, n + _nb)),
        ],
        out_specs=pl.BlockSpec((bc, 128), lambda n: (n, 0)),
        out_shape=jax.ShapeDtypeStruct((half, 128), jnp.float32),
    )(table_t, table_t)


def _gather_work(work, bufs, gsems, wsems):
    pending = [None, None]
    for k, (tab, ixv, out, base, j) in enumerate(work):
        slot = k % 2
        if pending[slot] is not None:
            pending[slot].wait()
            pending[slot] = None
        pltpu.async_copy(tab.at[ixv.at[j]], bufs[slot], gsems[slot]).wait()
        pending[slot] = pltpu.async_copy(
            bufs[slot], out.at[pl.ds(base + j * _CHUNK, _CHUNK)], wsems[slot])
    for p in pending:
        if p is not None:
            p.wait()


def _sc_gather_u_body(urow, utab, out_u, uix_v, rows_a, rows_b,
                      gsem_a, gsem_b, wsem_a, wsem_b):
    wid = lax.axis_index("s") * _NC + lax.axis_index("c")
    base = wid * _BPW
    pltpu.sync_copy(urow.at[pl.ds(wid * _NCHUNK, _NCHUNK)], uix_v)
    work = [(utab, uix_v, out_u, base, j) for j in range(_NCHUNK)]
    _gather_work(work, (rows_a, rows_b), (gsem_a, gsem_b), (wsem_a, wsem_b))


def _sc_gather_ig_body(irow, grow, itab, gtab, out_i, out_g,
                       iix_v, gix_v, rows_a, rows_b,
                       gsem_a, gsem_b, wsem_a, wsem_b):
    wid = lax.axis_index("s") * _NC + lax.axis_index("c")
    base = wid * _BPW
    pltpu.sync_copy(irow.at[pl.ds(wid * _NCHUNK, _NCHUNK)], iix_v)
    pltpu.sync_copy(grow.at[pl.ds(wid * _NCHUNK, _NCHUNK)], gix_v)
    work = [(itab, iix_v, out_i, base, j) for j in range(_NCHUNK)]
    work += [(gtab, gix_v, out_g, base, j) for j in range(_NCHUNK)]
    _gather_work(work, (rows_a, rows_b), (gsem_a, gsem_b), (wsem_a, wsem_b))


_SC_GATHER_U = functools.partial(
    pl.kernel,
    out_type=jax.ShapeDtypeStruct((B, 128), jnp.float32),
    mesh=plsc.VectorSubcoreMesh(core_axis_name="c", subcore_axis_name="s"),
    compiler_params=pltpu.CompilerParams(use_tc_tiling_on_sc=False),
    scratch_types=[
        pltpu.VMEM((_NCHUNK, _CHUNK), jnp.int32),
        pltpu.VMEM((_CHUNK, 128), jnp.float32),
        pltpu.VMEM((_CHUNK, 128), jnp.float32),
        pltpu.SemaphoreType.DMA,
        pltpu.SemaphoreType.DMA,
        pltpu.SemaphoreType.DMA,
        pltpu.SemaphoreType.DMA,
    ],
)(_sc_gather_u_body)


_SC_GATHER_IG = functools.partial(
    pl.kernel,
    out_type=[jax.ShapeDtypeStruct((B, 128), jnp.float32)] * 2,
    mesh=plsc.VectorSubcoreMesh(core_axis_name="c", subcore_axis_name="s"),
    compiler_params=pltpu.CompilerParams(use_tc_tiling_on_sc=False),
    scratch_types=[
        pltpu.VMEM((_NCHUNK, _CHUNK), jnp.int32),
        pltpu.VMEM((_NCHUNK, _CHUNK), jnp.int32),
        pltpu.VMEM((_CHUNK, 128), jnp.float32),
        pltpu.VMEM((_CHUNK, 128), jnp.float32),
        pltpu.SemaphoreType.DMA,
        pltpu.SemaphoreType.DMA,
        pltpu.SemaphoreType.DMA,
        pltpu.SemaphoreType.DMA,
    ],
)(_sc_gather_ig_body)


_BLK = 4096


def _mlp_body(u_ref, i_ref, g_ref, m_ref,
              w1_ref, b1_ref, w2t_ref, b2_ref, o_ref):
    m = m_ref[...]
    xu = jnp.where((m & 1) != 0, u_ref[:, 64:128], u_ref[:, 0:64])
    xi = jnp.where((m & 2) != 0, i_ref[:, 64:128], i_ref[:, 0:64])
    xg = jnp.where((m & 4) != 0, g_ref[:, 64:128], g_ref[:, 0:64])
    x = jnp.concatenate([xu, xi, xg], axis=1)
    h = jnp.dot(x, w1_ref[...], preferred_element_type=jnp.float32) + b1_ref[...]
    h = jnp.maximum(h, 0.0)
    o = lax.dot_general(h, w2t_ref[...], (((1,), (1,)), ((), ())),
                        preferred_element_type=jnp.float32) + b2_ref[...]
    o_ref[...] = jnp.maximum(o, 0.0).T


def _tc_mlp(u, i, g, m, W1, b1, W2t, b2):
    return pl.pallas_call(
        _mlp_body,
        grid=(B // _BLK,),
        in_specs=[
            pl.BlockSpec((_BLK, 128), lambda n: (n, 0)),
            pl.BlockSpec((_BLK, 128), lambda n: (n, 0)),
            pl.BlockSpec((_BLK, 128), lambda n: (n, 0)),
            pl.BlockSpec((_BLK, 1), lambda n: (n, 0)),
            pl.BlockSpec((IN_DIM, H1), lambda n: (0, 0)),
            pl.BlockSpec((1, H1), lambda n: (0, 0)),
            pl.BlockSpec((H2, H1), lambda n: (0, 0)),
            pl.BlockSpec((1, H2), lambda n: (0, 0)),
        ],
        out_specs=pl.BlockSpec((H2, _BLK), lambda n: (0, n)),
        out_shape=jax.ShapeDtypeStruct((H2, B), jnp.float32),
    )(u, i, g, m, W1, b1.reshape(1, H1), W2t, b2.reshape(1, H2))


def kernel(user_input, item_input, genre_input, user_table, item_table,
           genre_table, W1, b1, W2, b2):
    ip = _pack(item_table.T, _HALF_I, _BC_I)
    gp = _pack(genre_table.T, _HALF_G, _BC_G)
    ut, ip, gp = lax.optimization_barrier((user_table.T, ip, gp))
    up = _pack(ut, _HALF_U, _BC_U)

    urow = jnp.where(user_input >= _HALF_U, user_input - _HALF_U,
                     user_input).reshape(B // _CHUNK, _CHUNK)
    irow = jnp.where(item_input >= _HALF_I, item_input - _HALF_I,
                     item_input).reshape(B // _CHUNK, _CHUNK)
    grow = jnp.where(genre_input >= _HALF_G, genre_input - _HALF_G,
                     genre_input).reshape(B // _CHUNK, _CHUNK)
    m = ((user_input >= _HALF_U).astype(jnp.int32)
         + 2 * (item_input >= _HALF_I).astype(jnp.int32)
         + 4 * (genre_input >= _HALF_G).astype(jnp.int32)).reshape(B, 1)

    i, g = _SC_GATHER_IG(irow, grow, ip, gp)
    u = _SC_GATHER_U(urow, up)
    return _tc_mlp(u, i, g, m, W1, b1, W2.T, b2).T


# speedup vs baseline: 1.0620x; 1.0620x over previous
import functools

import jax
import jax.numpy as jnp
from jax import lax
from jax.experimental import pallas as pl
from jax.experimental.pallas import tpu as pltpu
from jax.experimental.pallas import tpu_sc as plsc

B = 16384
EMB = 64
IN_DIM = 3 * EMB
H1 = 128
H2 = 64

_HALF_U = 500480
_BC_U = 10880
_HALF_I = 51200
_BC_I = 10240
_HALF_G = 512
_BC_G = 512

_NC = 2
_NS = 16
_NW = _NC * _NS
_BPW = B // _NW
_CHUNK = 128
_NCHUNK = _BPW // _CHUNK


def _pack_body(l_ref, r_ref, o_ref):
    o_ref[...] = jnp.concatenate([l_ref[...].T, r_ref[...].T], axis=1)


def _pack(table_t, half, bc):
    nb = half // bc
    return pl.pallas_call(
        _pack_body,
        grid=(nb,),
        in_specs=[
            pl.BlockSpec((EMB, bc), lambda n: (0, n)),
            pl.BlockSpec((EMB, bc), lambda n, _nb=nb: (0, n + _nb)),
        ],
        out_specs=pl.BlockSpec((bc, 128), lambda n: (n, 0)),
        out_shape=jax.ShapeDtypeStruct((half, 128), jnp.float32),
    )(table_t, table_t)


def _gather_work(work, bufs, gsems, wsems):
    pending = [None, None]
    for k, (tab, ixv, out, base, j) in enumerate(work):
        slot = k % 2
        if pending[slot] is not None:
            pending[slot].wait()
            pending[slot] = None
        pltpu.async_copy(tab.at[ixv.at[j]], bufs[slot], gsems[slot]).wait()
        pending[slot] = pltpu.async_copy(
            bufs[slot], out.at[pl.ds(base + j * _CHUNK, _CHUNK)], wsems[slot])
    for p in pending:
        if p is not None:
            p.wait()


def _sc_gather_u_body(urow, utab, out_u, uix_v, rows_a, rows_b,
                      gsem_a, gsem_b, wsem_a, wsem_b):
    wid = lax.axis_index("s") * _NC + lax.axis_index("c")
    base = wid * _BPW
    pltpu.sync_copy(urow.at[pl.ds(wid * _NCHUNK, _NCHUNK)], uix_v)
    work = [(utab, uix_v, out_u, base, j) for j in range(_NCHUNK)]
    _gather_work(work, (rows_a, rows_b), (gsem_a, gsem_b), (wsem_a, wsem_b))


def _sc_gather_ig_body(irow, grow, itab, gtab, out_i, out_g,
                       iix_v, gix_v, rows_a, rows_b,
                       gsem_a, gsem_b, wsem_a, wsem_b):
    wid = lax.axis_index("s") * _NC + lax.axis_index("c")
    base = wid * _BPW
    pltpu.sync_copy(irow.at[pl.ds(wid * _NCHUNK, _NCHUNK)], iix_v)
    pltpu.sync_copy(grow.at[pl.ds(wid * _NCHUNK, _NCHUNK)], gix_v)
    work = [(itab, iix_v, out_i, base, j) for j in range(_NCHUNK)]
    work += [(gtab, gix_v, out_g, base, j) for j in range(_NCHUNK)]
    _gather_work(work, (rows_a, rows_b), (gsem_a, gsem_b), (wsem_a, wsem_b))


_SC_GATHER_U = functools.partial(
    pl.kernel,
    out_type=jax.ShapeDtypeStruct((B, 128), jnp.float32),
    mesh=plsc.VectorSubcoreMesh(core_axis_name="c", subcore_axis_name="s"),
    compiler_params=pltpu.CompilerParams(use_tc_tiling_on_sc=False),
    scratch_types=[
        pltpu.VMEM((_NCHUNK, _CHUNK), jnp.int32),
        pltpu.VMEM((_CHUNK, 128), jnp.float32),
        pltpu.VMEM((_CHUNK, 128), jnp.float32),
        pltpu.SemaphoreType.DMA,
        pltpu.SemaphoreType.DMA,
        pltpu.SemaphoreType.DMA,
        pltpu.SemaphoreType.DMA,
    ],
)(_sc_gather_u_body)


_SC_GATHER_IG = functools.partial(
    pl.kernel,
    out_type=[jax.ShapeDtypeStruct((B, 128), jnp.float32)] * 2,
    mesh=plsc.VectorSubcoreMesh(core_axis_name="c", subcore_axis_name="s"),
    compiler_params=pltpu.CompilerParams(use_tc_tiling_on_sc=False),
    scratch_types=[
        pltpu.VMEM((_NCHUNK, _CHUNK), jnp.int32),
        pltpu.VMEM((_NCHUNK, _CHUNK), jnp.int32),
        pltpu.VMEM((_CHUNK, 128), jnp.float32),
        pltpu.VMEM((_CHUNK, 128), jnp.float32),
        pltpu.SemaphoreType.DMA,
        pltpu.SemaphoreType.DMA,
        pltpu.SemaphoreType.DMA,
        pltpu.SemaphoreType.DMA,
    ],
)(_sc_gather_ig_body)


_BLK = 4096


def _mlp_body(u_ref, i_ref, g_ref, m_ref,
              w1_ref, b1_ref, w2t_ref, b2_ref, o_ref):
    m = m_ref[...]
    xu = jnp.where((m & 1) != 0, u_ref[:, 64:128], u_ref[:, 0:64])
    xi = jnp.where((m & 2) != 0, i_ref[:, 64:128], i_ref[:, 0:64])
    xg = jnp.where((m & 4) != 0, g_ref[:, 64:128], g_ref[:, 0:64])
    x = jnp.concatenate([xu, xi, xg], axis=1)
    h = jnp.dot(x, w1_ref[...], preferred_element_type=jnp.float32) + b1_ref[...]
    h = jnp.maximum(h, 0.0)
    o = lax.dot_general(h, w2t_ref[...], (((1,), (1,)), ((), ())),
                        preferred_element_type=jnp.float32) + b2_ref[...]
    o_ref[...] = jnp.maximum(o, 0.0).T


def _tc_mlp(u, i, g, m, W1, b1, W2t, b2):
    return pl.pallas_call(
        _mlp_body,
        grid=(B // _BLK,),
        in_specs=[
            pl.BlockSpec((_BLK, 128), lambda n: (n, 0)),
            pl.BlockSpec((_BLK, 128), lambda n: (n, 0)),
            pl.BlockSpec((_BLK, 128), lambda n: (n, 0)),
            pl.BlockSpec((_BLK, 1), lambda n: (n, 0)),
            pl.BlockSpec((IN_DIM, H1), lambda n: (0, 0)),
            pl.BlockSpec((1, H1), lambda n: (0, 0)),
            pl.BlockSpec((H2, H1), lambda n: (0, 0)),
            pl.BlockSpec((1, H2), lambda n: (0, 0)),
        ],
        out_specs=pl.BlockSpec((H2, _BLK), lambda n: (0, n)),
        out_shape=jax.ShapeDtypeStruct((H2, B), jnp.float32),
    )(u, i, g, m, W1, b1.reshape(1, H1), W2t, b2.reshape(1, H2))


def kernel(user_input, item_input, genre_input, user_table, item_table,
           genre_table, W1, b1, W2, b2):
    ip = _pack(item_table.T, _HALF_I, _BC_I)
    gp = _pack(genre_table.T, _HALF_G, _BC_G)
    ut, ip, gp = lax.optimization_barrier((user_table.T, ip, gp))
    up = _pack(ut, _HALF_U, _BC_U)

    urow = jnp.where(user_input >= _HALF_U, user_input - _HALF_U,
                     user_input).reshape(B // _CHUNK, _CHUNK)
    irow = jnp.where(item_input >= _HALF_I, item_input - _HALF_I,
                     item_input).reshape(B // _CHUNK, _CHUNK)
    grow = jnp.where(genre_input >= _HALF_G, genre_input - _HALF_G,
                     genre_input).reshape(B // _CHUNK, _CHUNK)
    m = ((user_input >= _HALF_U).astype(jnp.int32)
         + 2 * (item_input >= _HALF_I).astype(jnp.int32)
         + 4 * (genre_input >= _HALF_G).astype(jnp.int32)).reshape(B, 1)

    i, g = _SC_GATHER_IG(irow, grow, ip, gp)
    u = _SC_GATHER_U(urow, up)
    return _tc_mlp(u, i, g, m, W1, b1, W2.T, b2).T
